# initial kernel scaffold (unmeasured)
import jax
import jax.numpy as jnp
from jax import lax
from jax.experimental import pallas as pl
from jax.experimental.pallas import tpu as pltpu

N_DEV = 8


def kernel(x, w_mat):
    m_glob, k_shard = x.shape
    k_glob, n = w_mat.shape
    m_per = m_glob // N_DEV

    def body(x_ref, w_ref, out_ref, gx_ref, send_sems, recv_sems):
        my = lax.axis_index("i")

        def desc(j):
            return pltpu.make_async_remote_copy(
                src_ref=x_ref.at[pl.ds(j * m_per, m_per), :],
                dst_ref=gx_ref.at[my],
                send_sem=send_sems.at[j],
                recv_sem=recv_sems.at[my],
                device_id=(j,),
                device_id_type=pl.DeviceIdType.MESH,
            )

        def recv_desc(j):
            return pltpu.make_async_remote_copy(
                src_ref=gx_ref.at[j],
                dst_ref=gx_ref.at[j],
                send_sem=send_sems.at[j],
                recv_sem=recv_sems.at[j],
                device_id=(j,),
                device_id_type=pl.DeviceIdType.MESH,
            )

        gx_ref[my] = x_ref[pl.ds(my * m_per, m_per), :]

        for j in range(N_DEV):
            @pl.when(j != my)
            def _():
                desc(j).start()

        for j in range(N_DEV):
            @pl.when(j != my)
            def _():
                recv_desc(j).wait_recv()

        acc = jnp.zeros((m_per, n), jnp.float32)
        for k in range(N_DEV):
            acc += jnp.dot(
                gx_ref[k],
                w_ref[k * k_shard:(k + 1) * k_shard, :],
                preferred_element_type=jnp.float32,
            )
        out_ref[...] = jnp.maximum(acc, 0.0)

        for j in range(N_DEV):
            @pl.when(j != my)
            def _():
                desc(j).wait_send()

    return pl.pallas_call(
        body,
        out_shape=jax.ShapeDtypeStruct((m_per, n), jnp.float32),
        in_specs=[
            pl.BlockSpec(memory_space=pltpu.VMEM),
            pl.BlockSpec(memory_space=pltpu.VMEM),
        ],
        out_specs=pl.BlockSpec(memory_space=pltpu.VMEM),
        scratch_shapes=[
            pltpu.VMEM((N_DEV, m_per, k_shard), jnp.float32),
            pltpu.SemaphoreType.DMA((N_DEV,)),
            pltpu.SemaphoreType.DMA((N_DEV,)),
        ],
        compiler_params=pltpu.CompilerParams(collective_id=0),
    )(x, w_mat)


# baseline (device time: 16392 ns/iter reference)
import jax
import jax.numpy as jnp
from jax import lax
from jax.experimental import pallas as pl
from jax.experimental.pallas import tpu as pltpu

N_DEV = 8


def kernel(x, w_mat):
    m_glob, k_shard = x.shape
    k_glob, n = w_mat.shape
    m_per = m_glob // N_DEV

    def body(x_ref, w_ref, out_ref, gx_ref, send_sems, recv_sems):
        my = lax.axis_index("i")

        def desc(j):
            return pltpu.make_async_remote_copy(
                src_ref=x_ref.at[pl.ds(j * m_per, m_per), :],
                dst_ref=gx_ref.at[my],
                send_sem=send_sems.at[j],
                recv_sem=recv_sems.at[my],
                device_id=(j,),
                device_id_type=pl.DeviceIdType.MESH,
            )

        def recv_desc(j):
            return pltpu.make_async_remote_copy(
                src_ref=gx_ref.at[j],
                dst_ref=gx_ref.at[j],
                send_sem=send_sems.at[j],
                recv_sem=recv_sems.at[j],
                device_id=(j,),
                device_id_type=pl.DeviceIdType.MESH,
            )

        gx_ref[my] = x_ref[pl.ds(my * m_per, m_per), :]

        for j in range(N_DEV):
            @pl.when(j != my)
            def _():
                desc(j).start()

        for j in range(N_DEV):
            @pl.when(j != my)
            def _():
                recv_desc(j).wait_recv()

        acc = jnp.zeros((m_per, n), jnp.float32)
        for k in range(N_DEV):
            acc += jnp.dot(
                gx_ref[k],
                w_ref[k * k_shard:(k + 1) * k_shard, :],
                preferred_element_type=jnp.float32,
            )
        out_ref[...] = jnp.maximum(acc, 0.0)

        for j in range(N_DEV):
            @pl.when(j != my)
            def _():
                desc(j).wait_send()

    return pl.pallas_call(
        body,
        out_shape=jax.ShapeDtypeStruct((m_per, n), jnp.float32),
        in_specs=[
            pl.BlockSpec(memory_space=pltpu.VMEM),
            pl.BlockSpec(memory_space=pltpu.VMEM),
        ],
        out_specs=pl.BlockSpec(memory_space=pltpu.VMEM),
        scratch_shapes=[
            pltpu.VMEM((N_DEV, m_per, k_shard), jnp.float32),
            pltpu.SemaphoreType.DMA((N_DEV,)),
            pltpu.SemaphoreType.DMA((N_DEV,)),
        ],
    )(x, w_mat)


# device time: 15627 ns/iter; 1.0490x vs baseline; 1.0490x over previous
import numpy as np

import jax
import jax.numpy as jnp
from jax import lax
from jax.experimental import pallas as pl
from jax.experimental.pallas import tpu as pltpu

N_DEV = 8

_COORDS = np.array(
    [(0, 0, 0), (1, 0, 0), (1, 1, 0), (0, 1, 0),
     (0, 0, 1), (1, 0, 1), (1, 1, 1), (0, 1, 1)]
)

_PEERS = np.zeros((N_DEV, N_DEV - 1), dtype=np.int32)
for _p in range(N_DEV):
    _others = [q for q in range(N_DEV) if q != _p]
    _others.sort(key=lambda q: (int(np.abs(_COORDS[_p] - _COORDS[q]).sum()), q))
    _PEERS[_p] = _others


def kernel(x, w_mat):
    m_glob, k_shard = x.shape
    k_glob, n = w_mat.shape
    m_per = m_glob // N_DEV

    def body(x_ref, w_ref, peers_ref, out_ref, gx_ref, send_sems, recv_sems):
        my = lax.axis_index("i")
        peers = [peers_ref[my, s] for s in range(N_DEV - 1)]

        def send_desc(j):
            return pltpu.make_async_remote_copy(
                src_ref=x_ref.at[pl.ds(j * m_per, m_per), :],
                dst_ref=gx_ref.at[my],
                send_sem=send_sems.at[j],
                recv_sem=recv_sems.at[my],
                device_id=(j,),
                device_id_type=pl.DeviceIdType.MESH,
            )

        def recv_desc(j):
            return pltpu.make_async_remote_copy(
                src_ref=gx_ref.at[j],
                dst_ref=gx_ref.at[j],
                send_sem=send_sems.at[j],
                recv_sem=recv_sems.at[j],
                device_id=(j,),
                device_id_type=pl.DeviceIdType.MESH,
            )

        barrier_sem = pltpu.get_barrier_semaphore()
        for s in range(N_DEV - 1):
            pl.semaphore_signal(
                barrier_sem, inc=1,
                device_id=(peers[s],), device_id_type=pl.DeviceIdType.MESH,
            )
        pl.semaphore_wait(barrier_sem, N_DEV - 1)

        for s in reversed(range(N_DEV - 1)):
            send_desc(peers[s]).start()

        acc = jnp.dot(
            x_ref[pl.ds(my * m_per, m_per), :],
            w_ref[pl.ds(my * k_shard, k_shard), :],
            preferred_element_type=jnp.float32,
        )

        for s in range(N_DEV - 1):
            j = peers[s]
            recv_desc(j).wait_recv()
            acc += jnp.dot(
                gx_ref[j],
                w_ref[pl.ds(j * k_shard, k_shard), :],
                preferred_element_type=jnp.float32,
            )

        out_ref[...] = jnp.maximum(acc, 0.0)

        for s in range(N_DEV - 1):
            send_desc(peers[s]).wait_send()

    return pl.pallas_call(
        body,
        out_shape=jax.ShapeDtypeStruct((m_per, n), jnp.float32),
        in_specs=[
            pl.BlockSpec(memory_space=pltpu.VMEM),
            pl.BlockSpec(memory_space=pltpu.VMEM),
            pl.BlockSpec(memory_space=pltpu.SMEM),
        ],
        out_specs=pl.BlockSpec(memory_space=pltpu.VMEM),
        scratch_shapes=[
            pltpu.VMEM((N_DEV, m_per, k_shard), jnp.float32),
            pltpu.SemaphoreType.DMA((N_DEV,)),
            pltpu.SemaphoreType.DMA((N_DEV,)),
        ],
        compiler_params=pltpu.CompilerParams(collective_id=0),
    )(x, w_mat, jnp.asarray(_PEERS))


# device time: 12312 ns/iter; 1.3314x vs baseline; 1.2692x over previous
import numpy as np

import jax
import jax.numpy as jnp
from jax import lax
from jax.experimental import pallas as pl
from jax.experimental.pallas import tpu as pltpu

N_DEV = 8

_COORDS = np.array(
    [(0, 0, 0), (1, 0, 0), (1, 1, 0), (0, 1, 0),
     (0, 0, 1), (1, 0, 1), (1, 1, 1), (0, 1, 1)]
)

_PEERS = np.zeros((N_DEV, N_DEV - 1), dtype=np.int32)
for _p in range(N_DEV):
    _others = [q for q in range(N_DEV) if q != _p]
    _others.sort(key=lambda q: (int(np.abs(_COORDS[_p] - _COORDS[q]).sum()), q))
    _PEERS[_p] = _others


def kernel(x, w_mat):
    m_glob, k_shard = x.shape
    k_glob, n = w_mat.shape
    m_per = m_glob // N_DEV

    def body(x_ref, w_ref, peers_ref, out_ref, xbf_ref, gx_ref,
             send_sems, recv_sems):
        my = lax.axis_index("i")
        peers = [peers_ref[my, s] for s in range(N_DEV - 1)]

        xbf_ref[...] = x_ref[...].astype(jnp.bfloat16)

        def send_desc(j):
            return pltpu.make_async_remote_copy(
                src_ref=xbf_ref.at[pl.ds(j * m_per, m_per), :],
                dst_ref=gx_ref.at[my],
                send_sem=send_sems.at[j],
                recv_sem=recv_sems.at[my],
                device_id=(j,),
                device_id_type=pl.DeviceIdType.MESH,
            )

        def recv_desc(j):
            return pltpu.make_async_remote_copy(
                src_ref=gx_ref.at[j],
                dst_ref=gx_ref.at[j],
                send_sem=send_sems.at[j],
                recv_sem=recv_sems.at[j],
                device_id=(j,),
                device_id_type=pl.DeviceIdType.MESH,
            )

        barrier_sem = pltpu.get_barrier_semaphore()
        for s in range(N_DEV - 1):
            pl.semaphore_signal(
                barrier_sem, inc=1,
                device_id=(peers[s],), device_id_type=pl.DeviceIdType.MESH,
            )
        pl.semaphore_wait(barrier_sem, N_DEV - 1)

        for s in reversed(range(N_DEV - 1)):
            send_desc(peers[s]).start()

        acc = jnp.dot(
            x_ref[pl.ds(my * m_per, m_per), :],
            w_ref[pl.ds(my * k_shard, k_shard), :],
            preferred_element_type=jnp.float32,
        )

        for s in range(N_DEV - 1):
            j = peers[s]
            recv_desc(j).wait_recv()
            acc += jnp.dot(
                gx_ref[j].astype(jnp.float32),
                w_ref[pl.ds(j * k_shard, k_shard), :],
                preferred_element_type=jnp.float32,
            )

        out_ref[...] = jnp.maximum(acc, 0.0)

        for s in range(N_DEV - 1):
            send_desc(peers[s]).wait_send()

    return pl.pallas_call(
        body,
        out_shape=jax.ShapeDtypeStruct((m_per, n), jnp.float32),
        in_specs=[
            pl.BlockSpec(memory_space=pltpu.VMEM),
            pl.BlockSpec(memory_space=pltpu.VMEM),
            pl.BlockSpec(memory_space=pltpu.SMEM),
        ],
        out_specs=pl.BlockSpec(memory_space=pltpu.VMEM),
        scratch_shapes=[
            pltpu.VMEM((m_glob, k_shard), jnp.bfloat16),
            pltpu.VMEM((N_DEV, m_per, k_shard), jnp.bfloat16),
            pltpu.SemaphoreType.DMA((N_DEV,)),
            pltpu.SemaphoreType.DMA((N_DEV,)),
        ],
        compiler_params=pltpu.CompilerParams(collective_id=0),
    )(x, w_mat, jnp.asarray(_PEERS))
